# baseline (speedup 1.0000x reference)
import jax
import jax.numpy as jnp
from jax import lax
from jax.experimental import pallas as pl
from jax.experimental.pallas import tpu as pltpu
from jax.experimental.pallas import tpu_sc as plsc

_N = 10000
_E = 320000
_D = 128
_H = 64
_NG = 16
_NCLS = 6

_NCORES = 2
_NSUB = 16
_NW = _NCORES * _NSUB
_CHUNK = 128
_CHUNKS_PER_TILE = 80
_EDGES_PER_TILE = _CHUNK * _CHUNKS_PER_TILE
_EP = _NW * _EDGES_PER_TILE
_NPAD = 10240
_ROWS_PER_TILE = _NPAD // _NSUB

_vector_mesh = plsc.VectorSubcoreMesh(core_axis_name="c", subcore_axis_name="s")
_sc_params = pltpu.CompilerParams(use_tc_tiling_on_sc=False)


def _deg_body(dst_hbm, ones_hbm, zeros_hbm, out_hbm, ones_v, dst_v, dacc):
    cid = lax.axis_index("c")
    sid = lax.axis_index("s")
    wid = cid * _NSUB + sid
    row0 = sid * _ROWS_PER_TILE
    pltpu.sync_copy(zeros_hbm.at[pl.ds(row0, _ROWS_PER_TILE)],
                    dacc.at[pl.ds(row0, _ROWS_PER_TILE)])
    pltpu.sync_copy(ones_hbm, ones_v)
    pltpu.sync_copy(dst_hbm.at[wid], dst_v)
    plsc.subcore_barrier()

    @pl.loop(0, _CHUNKS_PER_TILE)
    def _chunk(j):
        pltpu.sync_copy(ones_v, dacc.at[dst_v.at[j]], add=True)

    plsc.subcore_barrier()
    pltpu.sync_copy(dacc.at[pl.ds(row0, _ROWS_PER_TILE)],
                    out_hbm.at[cid, pl.ds(row0, _ROWS_PER_TILE)])


def _sc_degree_count(dstp, ones, zeros16):
    kern = pl.kernel(
        _deg_body,
        out_type=jax.ShapeDtypeStruct((_NCORES, _NPAD, 16), jnp.float32),
        mesh=_vector_mesh,
        compiler_params=_sc_params,
        scratch_types=[
            pltpu.VMEM((_CHUNK, 16), jnp.float32),
            pltpu.VMEM((_CHUNKS_PER_TILE, _CHUNK), jnp.int32),
            pltpu.VMEM_SHARED((_NPAD, 16), jnp.float32),
        ],
    )
    return kern(dstp, ones, zeros16)


_DEPTH = 8


def _scatter_body(h_hbm, src_hbm, dst_hbm, zeros_hbm, out_hbm,
                  src_v, dst_v, rows, sems, acc):
    cid = lax.axis_index("c")
    sid = lax.axis_index("s")
    wid = cid * _NSUB + sid
    row0 = sid * _ROWS_PER_TILE
    pltpu.sync_copy(zeros_hbm.at[pl.ds(row0, _ROWS_PER_TILE)],
                    acc.at[pl.ds(row0, _ROWS_PER_TILE)])
    pltpu.sync_copy(src_hbm.at[wid], src_v)
    pltpu.sync_copy(dst_hbm.at[wid], dst_v)
    plsc.subcore_barrier()

    for b in range(_DEPTH):
        pltpu.async_copy(h_hbm.at[src_v.at[b]], rows.at[b], sems.at[b])

    @pl.loop(0, _CHUNKS_PER_TILE, step=_DEPTH)
    def _chunk(j):
        for b in range(_DEPTH):
            pltpu.make_async_copy(
                h_hbm.at[src_v.at[j + b]], rows.at[b], sems.at[b]).wait()
            pltpu.sync_copy(rows.at[b], acc.at[dst_v.at[j + b]], add=True)

            @pl.when(j + b + _DEPTH < _CHUNKS_PER_TILE)
            def _():
                pltpu.async_copy(h_hbm.at[src_v.at[j + b + _DEPTH]],
                                 rows.at[b], sems.at[b])

    plsc.subcore_barrier()
    pltpu.sync_copy(acc.at[pl.ds(row0, _ROWS_PER_TILE)],
                    out_hbm.at[cid, pl.ds(row0, _ROWS_PER_TILE)])


def _sc_scatter_rows(h, srcp, dstp, zeros64):
    kern = pl.kernel(
        _scatter_body,
        out_type=jax.ShapeDtypeStruct((_NCORES, _NPAD, _H), jnp.float32),
        mesh=_vector_mesh,
        compiler_params=_sc_params,
        scratch_types=[
            pltpu.VMEM((_CHUNKS_PER_TILE, _CHUNK), jnp.int32),
            pltpu.VMEM((_CHUNKS_PER_TILE, _CHUNK), jnp.int32),
            pltpu.VMEM((_DEPTH, _CHUNK, _H), jnp.float32),
            pltpu.SemaphoreType.DMA((_DEPTH,)),
            pltpu.VMEM_SHARED((_NPAD, _H), jnp.float32),
        ],
    )
    return kern(h, srcp, dstp, zeros64)


def _dinv_from_degp(degp_ref):
    deg = degp_ref[0, : _N, :] + degp_ref[1, : _N, :] + 1.0
    return lax.rsqrt(deg)[:, 0:1]


def _tc_first_body(x_ref, w1_ref, degp_ref, hs_ref):
    dinv = _dinv_from_degp(degp_ref)
    h = jnp.dot(x_ref[...], w1_ref[...], preferred_element_type=jnp.float32)
    hs_ref[...] = h * dinv


def _tc_first(x, W1, degp):
    return pl.pallas_call(
        _tc_first_body,
        out_shape=jax.ShapeDtypeStruct((_N, _H), jnp.float32),
    )(x, W1, degp)


def _tc_mid_body(s_ref, hs_ref, degp_ref, b1_ref, w2_ref, out_ref):
    dinv = _dinv_from_degp(degp_ref)
    s = s_ref[0, : _N, :] + s_ref[1, : _N, :] + hs_ref[...]
    x2 = jnp.maximum(s * dinv + b1_ref[...][None, :], 0.0)
    h2 = jnp.dot(x2, w2_ref[...], preferred_element_type=jnp.float32)
    out_ref[...] = h2 * dinv


def _tc_mid(s1, hs1, degp, b1, W2):
    return pl.pallas_call(
        _tc_mid_body,
        out_shape=jax.ShapeDtypeStruct((_N, _H), jnp.float32),
    )(s1, hs1, degp, b1, W2)


def _tc_head_body(s_ref, hs_ref, degp_ref, b2_ref, batch_ref,
                  wp_ref, bp_ref, wc_ref, bc_ref, logits_ref, z_ref):
    dinv = _dinv_from_degp(degp_ref)
    s = s_ref[0, : _N, :] + s_ref[1, : _N, :] + hs_ref[...]
    h = jnp.maximum(s * dinv + b2_ref[...][None, :], 0.0)
    gids = lax.broadcasted_iota(jnp.int32, (_NG, _N), 0)
    onehot = (gids == batch_ref[...][None, :]).astype(jnp.float32)
    sums = jnp.dot(onehot, h, preferred_element_type=jnp.float32)
    counts = jnp.sum(onehot, axis=1, keepdims=True)
    g = sums / jnp.maximum(counts, 1.0)
    z = jnp.dot(g, wp_ref[...], preferred_element_type=jnp.float32) \
        + bp_ref[...][None, :]
    logits_ref[...] = jnp.dot(z, wc_ref[...],
                              preferred_element_type=jnp.float32) \
        + bc_ref[...][None, :]
    z_ref[...] = z


def _tc_head(s2, hs2, degp, b2, batch, Wp, bp, Wc, bc):
    return pl.pallas_call(
        _tc_head_body,
        out_shape=(
            jax.ShapeDtypeStruct((_NG, _NCLS), jnp.float32),
            jax.ShapeDtypeStruct((_NG, _H), jnp.float32),
        ),
    )(s2, hs2, degp, b2, batch, Wp, bp, Wc, bc)


def kernel(x, edge_index, batch, W1, b1, W2, b2, Wp, bp, Wc, bc):
    npad_e = _EP - _E
    pad_src = jnp.zeros((npad_e,), jnp.int32)
    pad_dst = _N + (jnp.arange(npad_e, dtype=jnp.int32) % (_NPAD - _N))
    srcp = jnp.concatenate([edge_index[0], pad_src]).reshape(
        _NW, _CHUNKS_PER_TILE, _CHUNK)
    dstp = jnp.concatenate([edge_index[1], pad_dst]).reshape(
        _NW, _CHUNKS_PER_TILE, _CHUNK)
    ones16 = jnp.ones((_CHUNK, 16), jnp.float32)
    zeros16 = jnp.zeros((_NPAD, 16), jnp.float32)
    zeros64 = jnp.zeros((_NPAD, _H), jnp.float32)

    degp = _sc_degree_count(dstp, ones16, zeros16)
    hs1 = _tc_first(x, W1, degp)
    s1 = _sc_scatter_rows(hs1, srcp, dstp, zeros64)
    hs2 = _tc_mid(s1, hs1, degp, b1, W2)
    s2 = _sc_scatter_rows(hs2, srcp, dstp, zeros64)
    logits, z = _tc_head(s2, hs2, degp, b2, batch, Wp, bp, Wc, bc)
    return (logits, z)


# speedup vs baseline: 2.9232x; 2.9232x over previous
import jax
import jax.numpy as jnp
from jax import lax
from jax.experimental import pallas as pl
from jax.experimental.pallas import tpu as pltpu
from jax.experimental.pallas import tpu_sc as plsc

_N = 10000
_E = 320000
_D = 128
_H = 64
_NG = 16
_NCLS = 6

_NCORES = 2
_NSUB = 16
_NW = _NCORES * _NSUB
_CHUNK = 128
_CHUNKS_PER_TILE = 80
_EDGES_PER_TILE = _CHUNK * _CHUNKS_PER_TILE
_EP = _NW * _EDGES_PER_TILE
_NPAD = 10240
_ROWS_PER_TILE = _NPAD // _NSUB

_vector_mesh = plsc.VectorSubcoreMesh(core_axis_name="c", subcore_axis_name="s")
_sc_params = pltpu.CompilerParams(use_tc_tiling_on_sc=False)
_sc_vec_params = pltpu.CompilerParams(use_tc_tiling_on_sc=False,
                                      needs_layout_passes=False)


def _deg_body(dst_hbm, out_hbm, dst_v, hist):
    cid = lax.axis_index("c")
    sid = lax.axis_index("s")
    wid = cid * _NSUB + sid
    zeros16r = jnp.zeros((16,), jnp.float32)
    ones16r = jnp.ones((16,), jnp.float32)

    @pl.loop(0, _NPAD // 16)
    def _zero(i):
        hist[pl.ds(i * 16, 16)] = zeros16r

    pltpu.sync_copy(
        dst_hbm.at[pl.ds(wid * _CHUNKS_PER_TILE, _CHUNKS_PER_TILE)], dst_v)

    @pl.loop(0, _CHUNKS_PER_TILE)
    def _chunk(j):
        @pl.loop(0, _CHUNK // 16)
        def _grp(k):
            vidx = dst_v[j, pl.ds(k * 16, 16)]
            plsc.addupdate_scatter(hist, [vidx], ones16r)

    pltpu.sync_copy(hist, out_hbm.at[wid])


def _sc_degree_count(dstp):
    kern = pl.kernel(
        _deg_body,
        out_type=jax.ShapeDtypeStruct((_NW, _NPAD), jnp.float32),
        mesh=_vector_mesh,
        compiler_params=_sc_vec_params,
        scratch_types=[
            pltpu.VMEM((_CHUNKS_PER_TILE, _CHUNK), jnp.int32),
            pltpu.VMEM((_NPAD,), jnp.float32),
        ],
    )
    return kern(dstp)


_DEPTH = 8


_TCH = _NSUB * 2 * _CHUNKS_PER_TILE


def _gather_scatter_ring(h_hbm, src_v, dst_v, rows, sems, acc, nchunks):
    for b in range(_DEPTH):
        pltpu.async_copy(h_hbm.at[src_v.at[b]], rows.at[b], sems.at[b])

    @pl.loop(0, nchunks, step=_DEPTH)
    def _chunk(j):
        for b in range(_DEPTH):
            pltpu.make_async_copy(
                h_hbm.at[src_v.at[j + b]], rows.at[b], sems.at[b]).wait()
            pltpu.sync_copy(rows.at[b], acc.at[dst_v.at[j + b]], add=True)

            @pl.when(j + b + _DEPTH < nchunks)
            def _():
                pltpu.async_copy(h_hbm.at[src_v.at[j + b + _DEPTH]],
                                 rows.at[b], sems.at[b])


def _scatter_body(h_hbm, src_hbm, dst_hbm, zeros_hbm, out_hbm,
                  src_v, dst_v, rows, sems, acc):
    cid = lax.axis_index("c")
    sid = lax.axis_index("s")
    wid = cid * _NSUB + sid
    row0 = sid * _ROWS_PER_TILE
    pltpu.sync_copy(zeros_hbm.at[pl.ds(row0, _ROWS_PER_TILE)],
                    acc.at[pl.ds(row0, _ROWS_PER_TILE)])
    base = wid * _CHUNKS_PER_TILE
    pltpu.sync_copy(src_hbm.at[pl.ds(base, _CHUNKS_PER_TILE)], src_v)
    pltpu.sync_copy(dst_hbm.at[pl.ds(base, _CHUNKS_PER_TILE)], dst_v)
    plsc.subcore_barrier()
    _gather_scatter_ring(h_hbm, src_v, dst_v, rows, sems, acc,
                         _CHUNKS_PER_TILE)
    plsc.subcore_barrier()
    pltpu.sync_copy(acc.at[pl.ds(row0, _ROWS_PER_TILE)],
                    out_hbm.at[cid, pl.ds(row0, _ROWS_PER_TILE)])


_sc_scatter_rows = pl.kernel(
    _scatter_body,
    out_type=jax.ShapeDtypeStruct((_NCORES, _NPAD, _H), jnp.float32),
    mesh=_vector_mesh,
    compiler_params=_sc_params,
    scratch_types=[
        pltpu.VMEM((_CHUNKS_PER_TILE, _CHUNK), jnp.int32),
        pltpu.VMEM((_CHUNKS_PER_TILE, _CHUNK), jnp.int32),
        pltpu.VMEM((_DEPTH, _CHUNK, _H), jnp.float32),
        pltpu.SemaphoreType.DMA((_DEPTH,)),
        pltpu.VMEM_SHARED((_NPAD, _H), jnp.float32),
    ],
)


def _dinv_from_degp(degp_ref):
    ones_w = jnp.ones((_NW, 1), jnp.float32)
    deg = lax.dot_general(degp_ref[...], ones_w, (((0,), (0,)), ((), ())),
                          preferred_element_type=jnp.float32)
    return lax.rsqrt(deg[: _N, :] + 1.0)


def _tc_first_body(x_ref, w1_ref, degp_ref, hs_ref):
    dinv = _dinv_from_degp(degp_ref)
    h = jnp.dot(x_ref[...], w1_ref[...], preferred_element_type=jnp.float32)
    hs_ref[...] = h * dinv


def _tc_first(x, W1, degp):
    return pl.pallas_call(
        _tc_first_body,
        out_shape=jax.ShapeDtypeStruct((_N, _H), jnp.float32),
    )(x, W1, degp)


def _tc_mid_body(s_ref, hs_ref, degp_ref, b1_ref, w2_ref, out_ref):
    dinv = _dinv_from_degp(degp_ref)
    s = s_ref[0, : _N, :] + s_ref[1, : _N, :] + hs_ref[...]
    x2 = jnp.maximum(s * dinv + b1_ref[...][None, :], 0.0)
    h2 = jnp.dot(x2, w2_ref[...], preferred_element_type=jnp.float32)
    out_ref[...] = h2 * dinv


def _tc_mid(s1, hs1, degp, b1, W2):
    return pl.pallas_call(
        _tc_mid_body,
        out_shape=jax.ShapeDtypeStruct((_N, _H), jnp.float32),
    )(s1, hs1, degp, b1, W2)


def _tc_head_body(s_ref, hs_ref, degp_ref, b2_ref, batch_ref,
                  wp_ref, bp_ref, wc_ref, bc_ref, logits_ref, z_ref):
    dinv = _dinv_from_degp(degp_ref)
    s = s_ref[0, : _N, :] + s_ref[1, : _N, :] + hs_ref[...]
    h = jnp.maximum(s * dinv + b2_ref[...][None, :], 0.0)
    gids = lax.broadcasted_iota(jnp.int32, (_NG, _N), 0)
    onehot = (gids == batch_ref[...][None, :]).astype(jnp.float32)
    sums = jnp.dot(onehot, h, preferred_element_type=jnp.float32)
    counts = jnp.sum(onehot, axis=1, keepdims=True)
    g = sums / jnp.maximum(counts, 1.0)
    z = jnp.dot(g, wp_ref[...], preferred_element_type=jnp.float32) \
        + bp_ref[...][None, :]
    logits_ref[...] = jnp.dot(z, wc_ref[...],
                              preferred_element_type=jnp.float32) \
        + bc_ref[...][None, :]
    z_ref[...] = z


def _tc_head(s2, hs2, degp, b2, batch, Wp, bp, Wc, bc):
    return pl.pallas_call(
        _tc_head_body,
        out_shape=(
            jax.ShapeDtypeStruct((_NG, _NCLS), jnp.float32),
            jax.ShapeDtypeStruct((_NG, _H), jnp.float32),
        ),
    )(s2, hs2, degp, b2, batch, Wp, bp, Wc, bc)


def kernel(x, edge_index, batch, W1, b1, W2, b2, Wp, bp, Wc, bc):
    npad_e = _TCH * _CHUNK - _E
    pad_src = jnp.arange(npad_e, dtype=jnp.int32) % _N
    pad_dst = _N + (jnp.arange(npad_e, dtype=jnp.int32) % (_NPAD - _N))
    srcp = jnp.concatenate([edge_index[0], pad_src]).reshape(_TCH, _CHUNK)
    dstp = jnp.concatenate([edge_index[1], pad_dst]).reshape(_TCH, _CHUNK)
    zeros64 = jnp.zeros((_NPAD, _H), jnp.float32)

    degp = _sc_degree_count(dstp)
    hs1 = _tc_first(x, W1, degp)
    s1 = _sc_scatter_rows(hs1, srcp, dstp, zeros64)
    hs2 = _tc_mid(s1, hs1, degp, b1, W2)
    s2 = _sc_scatter_rows(hs2, srcp, dstp, zeros64)
    logits, z = _tc_head(s2, hs2, degp, b2, batch, Wp, bp, Wc, bc)
    return (logits, z)


# speedup vs baseline: 2.9298x; 1.0023x over previous
import jax
import jax.numpy as jnp
from jax import lax
from jax.experimental import pallas as pl
from jax.experimental.pallas import tpu as pltpu
from jax.experimental.pallas import tpu_sc as plsc

_N = 10000
_E = 320000
_D = 128
_H = 64
_NG = 16
_NCLS = 6

_NCORES = 2
_NSUB = 16
_NW = _NCORES * _NSUB
_CHUNK = 128
_CHUNKS_PER_TILE = 80
_EDGES_PER_TILE = _CHUNK * _CHUNKS_PER_TILE
_EP = _NW * _EDGES_PER_TILE
_NPAD = 10240
_ROWS_PER_TILE = _NPAD // _NSUB

_vector_mesh = plsc.VectorSubcoreMesh(core_axis_name="c", subcore_axis_name="s")
_sc_params = pltpu.CompilerParams(use_tc_tiling_on_sc=False)
_sc_vec_params = pltpu.CompilerParams(use_tc_tiling_on_sc=False,
                                      needs_layout_passes=False)


def _deg_body(dst_hbm, out_hbm, dst_v, hist):
    cid = lax.axis_index("c")
    sid = lax.axis_index("s")
    wid = cid * _NSUB + sid
    zeros16r = jnp.zeros((16,), jnp.float32)
    ones16r = jnp.ones((16,), jnp.float32)

    @pl.loop(0, _NPAD // 16)
    def _zero(i):
        hist[pl.ds(i * 16, 16)] = zeros16r

    pltpu.sync_copy(
        dst_hbm.at[pl.ds(wid * _CHUNKS_PER_TILE, _CHUNKS_PER_TILE)], dst_v)

    @pl.loop(0, _CHUNKS_PER_TILE)
    def _chunk(j):
        @pl.loop(0, _CHUNK // 16)
        def _grp(k):
            vidx = dst_v[j, pl.ds(k * 16, 16)]
            plsc.addupdate_scatter(hist, [vidx], ones16r)

    pltpu.sync_copy(hist, out_hbm.at[wid])


def _sc_degree_count(dstp):
    kern = pl.kernel(
        _deg_body,
        out_type=jax.ShapeDtypeStruct((_NW, _NPAD), jnp.float32),
        mesh=_vector_mesh,
        compiler_params=_sc_vec_params,
        scratch_types=[
            pltpu.VMEM((_CHUNKS_PER_TILE, _CHUNK), jnp.int32),
            pltpu.VMEM((_NPAD,), jnp.float32),
        ],
    )
    return kern(dstp)


_DEPTH = 8


_TCH = _NSUB * 2 * _CHUNKS_PER_TILE


def _gather_scatter_ring(h_hbm, src_v, dst_v, rows, sems, acc, nchunks):
    for b in range(_DEPTH):
        pltpu.async_copy(h_hbm.at[src_v.at[b]], rows.at[b], sems.at[b])

    @pl.loop(0, nchunks, step=_DEPTH)
    def _chunk(j):
        for b in range(_DEPTH):
            pltpu.make_async_copy(
                h_hbm.at[src_v.at[j + b]], rows.at[b], sems.at[b]).wait()
            pltpu.sync_copy(rows.at[b], acc.at[dst_v.at[j + b]], add=True)

            @pl.when(j + b + _DEPTH < nchunks)
            def _():
                pltpu.async_copy(h_hbm.at[src_v.at[j + b + _DEPTH]],
                                 rows.at[b], sems.at[b])


def _scatter_body(h_hbm, src_hbm, dst_hbm, zeros_hbm, out_hbm,
                  src_v, dst_v, rows, sems, acc):
    cid = lax.axis_index("c")
    sid = lax.axis_index("s")
    wid = cid * _NSUB + sid
    row0 = sid * _ROWS_PER_TILE
    pltpu.sync_copy(zeros_hbm.at[pl.ds(row0, _ROWS_PER_TILE)],
                    acc.at[pl.ds(row0, _ROWS_PER_TILE)])
    base = wid * _CHUNKS_PER_TILE
    pltpu.sync_copy(src_hbm.at[pl.ds(base, _CHUNKS_PER_TILE)], src_v)
    pltpu.sync_copy(dst_hbm.at[pl.ds(base, _CHUNKS_PER_TILE)], dst_v)
    plsc.subcore_barrier()
    _gather_scatter_ring(h_hbm, src_v, dst_v, rows, sems, acc,
                         _CHUNKS_PER_TILE)
    plsc.subcore_barrier()
    pltpu.sync_copy(acc.at[pl.ds(row0, _ROWS_PER_TILE)],
                    out_hbm.at[cid, pl.ds(row0, _ROWS_PER_TILE)])


_sc_scatter_rows = pl.kernel(
    _scatter_body,
    out_type=jax.ShapeDtypeStruct((_NCORES, _NPAD, _H), jnp.float32),
    mesh=_vector_mesh,
    compiler_params=_sc_params,
    scratch_types=[
        pltpu.VMEM((_CHUNKS_PER_TILE, _CHUNK), jnp.int32),
        pltpu.VMEM((_CHUNKS_PER_TILE, _CHUNK), jnp.int32),
        pltpu.VMEM((_DEPTH, _CHUNK, _H), jnp.float32),
        pltpu.SemaphoreType.DMA((_DEPTH,)),
        pltpu.VMEM_SHARED((_NPAD, _H), jnp.float32),
    ],
)


def _dinv_from_degp(degp_ref):
    ones_w = jnp.ones((_NW, 1), jnp.float32)
    deg = lax.dot_general(degp_ref[...], ones_w, (((0,), (0,)), ((), ())),
                          preferred_element_type=jnp.float32)
    return lax.rsqrt(deg[: _N, :] + 1.0)


def _tc_first_body(x_ref, w1_ref, degp_ref, hs_ref, dinv_ref):
    dinv = _dinv_from_degp(degp_ref)
    h = jnp.dot(x_ref[...], w1_ref[...], preferred_element_type=jnp.float32)
    hs_ref[...] = h * dinv
    dinv_ref[...] = dinv


def _tc_first(x, W1, degp):
    return pl.pallas_call(
        _tc_first_body,
        out_shape=(jax.ShapeDtypeStruct((_N, _H), jnp.float32),
                   jax.ShapeDtypeStruct((_N, 1), jnp.float32)),
    )(x, W1, degp)


def _tc_mid_body(s_ref, hs_ref, dinv_ref, b1_ref, w2_ref, out_ref):
    dinv = dinv_ref[...]
    s = s_ref[0, : _N, :] + s_ref[1, : _N, :] + hs_ref[...]
    x2 = jnp.maximum(s * dinv + b1_ref[...][None, :], 0.0)
    h2 = jnp.dot(x2, w2_ref[...], preferred_element_type=jnp.float32)
    out_ref[...] = h2 * dinv


def _tc_mid(s1, hs1, dinv, b1, W2):
    return pl.pallas_call(
        _tc_mid_body,
        out_shape=jax.ShapeDtypeStruct((_N, _H), jnp.float32),
    )(s1, hs1, dinv, b1, W2)


def _tc_head_body(s_ref, hs_ref, dinv_ref, b2_ref, batch_ref,
                  wp_ref, bp_ref, wc_ref, bc_ref, logits_ref, z_ref):
    dinv = dinv_ref[...]
    s = s_ref[0, : _N, :] + s_ref[1, : _N, :] + hs_ref[...]
    h = jnp.maximum(s * dinv + b2_ref[...][None, :], 0.0)
    gids = lax.broadcasted_iota(jnp.int32, (_NG, _N), 0)
    onehot = (gids == batch_ref[...][None, :]).astype(jnp.float32)
    sums = jnp.dot(onehot, h, preferred_element_type=jnp.float32)
    counts = jnp.sum(onehot, axis=1, keepdims=True)
    g = sums / jnp.maximum(counts, 1.0)
    z = jnp.dot(g, wp_ref[...], preferred_element_type=jnp.float32) \
        + bp_ref[...][None, :]
    logits_ref[...] = jnp.dot(z, wc_ref[...],
                              preferred_element_type=jnp.float32) \
        + bc_ref[...][None, :]
    z_ref[...] = z


def _tc_head(s2, hs2, dinv, b2, batch, Wp, bp, Wc, bc):
    return pl.pallas_call(
        _tc_head_body,
        out_shape=(
            jax.ShapeDtypeStruct((_NG, _NCLS), jnp.float32),
            jax.ShapeDtypeStruct((_NG, _H), jnp.float32),
        ),
    )(s2, hs2, dinv, b2, batch, Wp, bp, Wc, bc)


def kernel(x, edge_index, batch, W1, b1, W2, b2, Wp, bp, Wc, bc):
    npad_e = _TCH * _CHUNK - _E
    pad_src = jnp.arange(npad_e, dtype=jnp.int32) % _N
    pad_dst = _N + (jnp.arange(npad_e, dtype=jnp.int32) % (_NPAD - _N))
    srcp = jnp.concatenate([edge_index[0], pad_src]).reshape(_TCH, _CHUNK)
    dstp = jnp.concatenate([edge_index[1], pad_dst]).reshape(_TCH, _CHUNK)
    zeros64 = jnp.zeros((_NPAD, _H), jnp.float32)

    degp = _sc_degree_count(dstp)
    hs1, dinv = _tc_first(x, W1, degp)
    s1 = _sc_scatter_rows(hs1, srcp, dstp, zeros64)
    hs2 = _tc_mid(s1, hs1, dinv, b1, W2)
    s2 = _sc_scatter_rows(hs2, srcp, dstp, zeros64)
    logits, z = _tc_head(s2, hs2, dinv, b2, batch, Wp, bp, Wc, bc)
    return (logits, z)
